# baseline (speedup 1.0000x reference)
import functools

import jax
import jax.numpy as jnp
from jax import lax
from jax.experimental import pallas as pl
from jax.experimental.pallas import tpu as pltpu
from jax.experimental.pallas import tpu_sc as plsc

NUM_RELATIONS = 1000000
EMB_D = 32
BATCH_B = 4096
FIELDS_F = 26
TOTAL = BATCH_B * FIELDS_F

_INFO = plsc.get_sparse_core_info()
NC = _INFO.num_cores
NS = _INFO.num_subcores
NW = NC * NS

SH_LG = 15
CH_LG = 10
CHW = 1 << CH_LG
NCHUNK = 1 << (SH_LG - CH_LG)
IDS_PIECE = TOTAL // 8
SEL_CAP = 4128
CAP = 192
PROWS = CAP // 4
TCAP = 32
TAIL0 = 999936
DUMMY_ROW = TOTAL
OUT_ROWS = TOTAL + 128
DUMMY_ID = 0x7FFFFFFF
NSLOT = NW * NCHUNK + NW
SPW = NSLOT // NW

_MESH = plsc.VectorSubcoreMesh(core_axis_name="c", subcore_axis_name="s")


@functools.partial(
    pl.kernel,
    mesh=_MESH,
    compiler_params=pltpu.CompilerParams(needs_layout_passes=False),
    out_type=(
        jax.ShapeDtypeStruct((NSLOT * PROWS, 128), jnp.float32),
        jax.ShapeDtypeStruct((NSLOT * CAP,), jnp.int32),
    ),
    scratch_types=[
        pltpu.VMEM((IDS_PIECE,), jnp.int32),
        pltpu.VMEM((SEL_CAP,), jnp.int32),
        pltpu.VMEM((SEL_CAP,), jnp.int32),
        pltpu.VMEM((EMB_D, CHW), jnp.float32),
        pltpu.VMEM((EMB_D, CHW), jnp.float32),
        pltpu.VMEM((EMB_D, 128), jnp.float32),
        pltpu.VMEM((CAP + 16,), jnp.int32),
        pltpu.VMEM((CAP + 16,), jnp.int32),
        pltpu.VMEM((CAP + 16,), jnp.int32),
        pltpu.VMEM((PROWS, 128), jnp.float32),
        pltpu.VMEM((PROWS, 128), jnp.float32),
        pltpu.SemaphoreType.DMA,
        pltpu.SemaphoreType.DMA,
        pltpu.SemaphoreType.DMA,
        pltpu.SemaphoreType.DMA,
        pltpu.SemaphoreType.DMA,
    ],
)
def _scan(ids_hbm, table_hbm, tail_hbm, vals_hbm, pos_hbm,
          ids_v, sel_id, sel_pos, chk_a, chk_b, tail_v,
          cs_id, cp_a, cp_b, stg_a, stg_b,
          sem_ca, sem_cb, sem_t, sem_oa, sem_ob):
    wid = lax.axis_index("s") * NC + lax.axis_index("c")
    shard0 = lax.shift_left(wid, SH_LG)
    lane = lax.iota(jnp.int32, 16)

    def _fire_chunk(c, buf, sem):
        base = pl.multiple_of(shard0 + lax.shift_left(c, CH_LG), CHW)
        tail = base == (TAIL0 - 512)

        @pl.when(jnp.logical_not(tail) & (base + CHW <= NUM_RELATIONS))
        def _():
            pltpu.async_copy(table_hbm.at[:, pl.ds(base, CHW)], buf, sem)

        @pl.when(tail)
        def _():
            pltpu.async_copy(table_hbm.at[:, pl.ds(TAIL0 - 512, 512)],
                             buf.at[:, pl.ds(0, 512)], sem)
            pltpu.async_copy(table_hbm.at[:, pl.ds(0, 512)],
                             buf.at[:, pl.ds(512, 512)], sem)

        @pl.when(base + CHW > NUM_RELATIONS + CHW - 576)
        def _():
            pltpu.async_copy(table_hbm.at[:, pl.ds(0, CHW)], buf, sem)

    _fire_chunk(jnp.int32(0), chk_a, sem_ca)
    _fire_chunk(jnp.int32(1), chk_b, sem_cb)
    pltpu.async_copy(tail_hbm, tail_v, sem_t)

    def _piece(p, cnt):
        pltpu.sync_copy(ids_hbm.at[pl.ds(p * IDS_PIECE, IDS_PIECE)], ids_v)

        def _vec(v, cnt):
            idv = ids_v[pl.ds(v * 16, 16)]
            m = lax.shift_right_logical(idv, SH_LG) == wid
            posv = lane + (p * IDS_PIECE + v * 16)
            plsc.store_compressed(sel_id.at[pl.ds(cnt, 16)], idv, mask=m)
            plsc.store_compressed(sel_pos.at[pl.ds(cnt, 16)], posv, mask=m)
            return cnt + plsc.all_reduce_population_count(m)[0]

        return lax.fori_loop(0, IDS_PIECE // 16, _vec, cnt)

    cnt = lax.fori_loop(0, 8, _piece, jnp.int32(0))
    sel_id[pl.ds(cnt, 16)] = jnp.full((16,), DUMMY_ID, jnp.int32)
    sel_pos[pl.ds(cnt, 16)] = jnp.full((16,), DUMMY_ROW, jnp.int32)
    nvec = lax.shift_right_logical(cnt + 15, 4)

    def _prefill(cpos):
        zeros16 = jnp.zeros((16,), jnp.int32)
        dummy16 = jnp.full((16,), DUMMY_ROW, jnp.int32)
        for i in range((CAP + 16) // 16):
            cs_id[pl.ds(i * 16, 16)] = zeros16
            cpos[pl.ds(i * 16, 16)] = dummy16

    def _reclaim(cpos, stg, sem):
        pltpu.make_async_copy(
            vals_hbm.at[pl.ds(0, PROWS)], stg, sem).wait()
        pltpu.make_async_copy(
            pos_hbm.at[pl.ds(0, CAP)], cpos.at[pl.ds(0, CAP)], sem).wait()

    def _extract(src, nsel_groups, cpos, stg, colmask=CHW - 1):
        for g in range(nsel_groups):
            colv = jnp.bitwise_and(cs_id[pl.ds(g * 16, 16)], colmask)
            s16 = lane + g * 16
            srow = lax.shift_right_logical(s16, 2)
            sbase = lax.shift_left(jnp.bitwise_and(s16, 3), 5)
            for d in range(EMB_D):
                dv = jnp.full((16,), d, jnp.int32)
                val = plsc.load_gather(src, [dv, colv])
                plsc.store_scatter(stg, [srow, sbase + d], val)

    def _writeback(slot, cpos, stg, sem):
        pltpu.async_copy(stg, vals_hbm.at[pl.ds(slot * PROWS, PROWS)], sem)
        pltpu.async_copy(cpos.at[pl.ds(0, CAP)],
                         pos_hbm.at[pl.ds(slot * CAP, CAP)], sem)

    def _proc(c, cc, buf, sem_c, cpos, stg, sem_o):
        @pl.when(cc > 0)
        def _():
            _reclaim(cpos, stg, sem_o)

        _prefill(cpos)

        gch = lax.shift_left(wid, SH_LG - CH_LG) + c

        def _rvec(v, mc):
            idv = sel_id[pl.ds(v * 16, 16)]
            pv = sel_pos[pl.ds(v * 16, 16)]
            m = jnp.logical_and(
                lax.shift_right_logical(idv, CH_LG) == gch, idv < TAIL0)
            plsc.store_compressed(cs_id.at[pl.ds(mc, 16)], idv, mask=m)
            plsc.store_compressed(cpos.at[pl.ds(mc, 16)], pv, mask=m)
            return mc + plsc.all_reduce_population_count(m)[0]

        mc = lax.fori_loop(0, nvec, _rvec, jnp.int32(0))
        cs_id[pl.ds(mc, 16)] = jnp.zeros((16,), jnp.int32)
        cpos[pl.ds(mc, 16)] = jnp.full((16,), DUMMY_ROW, jnp.int32)

        pltpu.make_async_copy(
            table_hbm.at[:, pl.ds(0, CHW)], buf, sem_c).wait()
        _extract(buf, CAP // 16, cpos, stg)
        _writeback(lax.shift_left(wid, 5) + c, cpos, stg, sem_o)

    def _pair(cc, carry):
        c0 = cc * 2
        _proc(c0, cc, chk_a, sem_ca, cp_a, stg_a, sem_oa)

        @pl.when(c0 + 2 < NCHUNK)
        def _():
            _fire_chunk(c0 + 2, chk_a, sem_ca)

        _proc(c0 + 1, cc, chk_b, sem_cb, cp_b, stg_b, sem_ob)

        @pl.when(c0 + 3 < NCHUNK)
        def _():
            _fire_chunk(c0 + 3, chk_b, sem_cb)

        return carry

    lax.fori_loop(0, NCHUNK // 2, _pair, None)

    _reclaim(cp_a, stg_a, sem_oa)
    _prefill(cp_a)
    pltpu.make_async_copy(tail_hbm, tail_v, sem_t).wait()

    def _tvec(v, mc):
        idv = sel_id[pl.ds(v * 16, 16)]
        pv = sel_pos[pl.ds(v * 16, 16)]
        m = jnp.logical_and(idv >= TAIL0, idv < NUM_RELATIONS)
        plsc.store_compressed(cs_id.at[pl.ds(mc, 16)], idv, mask=m)
        plsc.store_compressed(cp_a.at[pl.ds(mc, 16)], pv, mask=m)
        return mc + plsc.all_reduce_population_count(m)[0]

    mct = lax.fori_loop(0, nvec, _tvec, jnp.int32(0))
    cs_id[pl.ds(mct, 16)] = jnp.zeros((16,), jnp.int32)
    cp_a[pl.ds(mct, 16)] = jnp.full((16,), DUMMY_ROW, jnp.int32)
    _extract(tail_v, TCAP // 16, cp_a, stg_a, colmask=127)
    _writeback(NW * NCHUNK + wid, cp_a, stg_a, sem_oa)
    _reclaim(cp_a, stg_a, sem_oa)
    _reclaim(cp_b, stg_b, sem_ob)


@functools.partial(
    pl.kernel,
    mesh=_MESH,
    compiler_params=pltpu.CompilerParams(
        use_tc_tiling_on_sc=False, needs_layout_passes=False),
    out_type=jax.ShapeDtypeStruct((OUT_ROWS, EMB_D), jnp.float32),
    scratch_types=[
        pltpu.VMEM((CAP,), jnp.int32),
        pltpu.VMEM((CAP // 32, 32), jnp.int32),
        pltpu.VMEM((CAP, EMB_D), jnp.float32),
        pltpu.VMEM((CAP, EMB_D), jnp.float32),
        pltpu.SemaphoreType.DMA,
        pltpu.SemaphoreType.DMA,
        pltpu.SemaphoreType.DMA,
        pltpu.SemaphoreType.DMA,
    ],
)
def _scatter(vals_hbm, pos_hbm, out_hbm, pv, pos2d, vb_a, vb_b,
             sem_la, sem_lb, sem_sa, sem_sb):
    wid = lax.axis_index("s") * NC + lax.axis_index("c")
    slot0 = wid * SPW

    bufs = (vb_a, vb_b)
    lsems = (sem_la, sem_lb)
    ssems = (sem_sa, sem_sb)

    for j in range(SPW):
        buf, lsem, ssem = bufs[j % 2], lsems[j % 2], ssems[j % 2]
        slot = slot0 + j
        pltpu.async_copy(vals_hbm.at[pl.ds(slot * CAP, CAP)], buf, lsem)
        pltpu.make_async_copy(
            vals_hbm.at[pl.ds(0, CAP)], buf, lsem).wait()
        pltpu.sync_copy(pos_hbm.at[pl.ds(slot * CAP, CAP)], pv)
        for k in range(CAP // 32):
            for i in range(2):
                pos2d[k, pl.ds(i * 16, 16)] = pv[pl.ds(k * 32 + i * 16, 16)]
        for k in range(CAP // 32):
            pltpu.async_copy(buf.at[pl.ds(k * 32, 32)],
                             out_hbm.at[pos2d.at[k]], ssem)
        for _ in range(CAP // 32):
            pltpu.make_async_copy(
                buf.at[pl.ds(0, 32)], out_hbm.at[pl.ds(0, 32)], ssem).wait()


def kernel(relation_ids, embedding_table):
    ids = relation_ids.astype(jnp.int32).reshape(TOTAL)
    table_t = embedding_table.T
    tail_t = jnp.pad(table_t[:, TAIL0:], ((0, 0), (0, 128 - (NUM_RELATIONS - TAIL0))))
    vals, pos = _scan(ids, table_t, tail_t)
    out = _scatter(vals.reshape(NSLOT * CAP, EMB_D), pos)
    return out[:TOTAL].reshape(BATCH_B, FIELDS_F, EMB_D)


# speedup vs baseline: 2.5124x; 2.5124x over previous
import functools

import jax
import jax.numpy as jnp
from jax import lax
from jax.experimental import pallas as pl
from jax.experimental.pallas import tpu as pltpu
from jax.experimental.pallas import tpu_sc as plsc

NUM_RELATIONS = 1000000
EMB_D = 32
BATCH_B = 4096
FIELDS_F = 26
TOTAL = BATCH_B * FIELDS_F

_INFO = plsc.get_sparse_core_info()
NC = _INFO.num_cores
NS = _INFO.num_subcores
NW = NC * NS
B_PER_W = TOTAL // NW
CHUNK = 128
NCHUNK = B_PER_W // CHUNK

_MESH = plsc.VectorSubcoreMesh(core_axis_name="c", subcore_axis_name="s")


@functools.partial(
    pl.kernel,
    mesh=_MESH,
    compiler_params=pltpu.CompilerParams(use_tc_tiling_on_sc=False),
    out_type=jax.ShapeDtypeStruct((TOTAL, EMB_D), jnp.float32),
    scratch_types=[
        pltpu.VMEM((NCHUNK, CHUNK), jnp.int32),
        pltpu.VMEM((B_PER_W, EMB_D), jnp.float32),
        pltpu.SemaphoreType.DMA,
    ],
)
def _gather_rows(idx_hbm, table_hbm, out_hbm, idx_v, rows_v, sem):
    wid = lax.axis_index("s") * NC + lax.axis_index("c")
    base = wid * B_PER_W
    pltpu.sync_copy(idx_hbm.at[wid], idx_v)
    copies = [
        pltpu.async_copy(
            table_hbm.at[idx_v.at[j]],
            rows_v.at[pl.ds(j * CHUNK, CHUNK)],
            sem,
        )
        for j in range(NCHUNK)
    ]
    for c in copies:
        c.wait()
    pltpu.sync_copy(rows_v, out_hbm.at[pl.ds(base, B_PER_W)])


def kernel(relation_ids, embedding_table):
    ids = relation_ids.astype(jnp.int32).reshape(NW, NCHUNK, CHUNK)
    out = _gather_rows(ids, embedding_table)
    return out.reshape(BATCH_B, FIELDS_F, EMB_D)
